# Initial kernel scaffold; baseline (speedup 1.0000x reference)
import jax
import jax.numpy as jnp
from jax.experimental import pallas as pl


def kernel(difference, epoch, iteration, weight, initial_weight):
    raise NotImplementedError("write your pallas kernel here")


# speedup vs baseline: 30.0429x; 30.0429x over previous
import jax
import jax.numpy as jnp
from jax.experimental import pallas as pl
from jax.experimental.pallas import tpu as pltpu

_TOP_K = 2000
_ROWS = 32
_COLS = 32768


def _topk_mask_kernel(iw_ref, x_ref, o_ref):
    x = x_ref[...]
    bits = jax.lax.bitcast_convert_type(x, jnp.int32) & jnp.int32(0x7FFFFFFF)

    def body(_, lohi):
        lo, hi = lohi
        d = hi - lo
        mid = lo + (d // 2) + (d & 1)
        cnt = jnp.sum((bits >= mid).astype(jnp.int32), axis=1, keepdims=True)
        ok = cnt >= _TOP_K
        return jnp.where(ok, mid, lo), jnp.where(ok, hi, mid - 1)

    lo0 = jnp.zeros((_ROWS, 1), jnp.int32)
    hi0 = jnp.full((_ROWS, 1), jnp.int32(0x7FFFFFFF))
    lo, _ = jax.lax.fori_loop(0, 31, body, (lo0, hi0))
    iw = iw_ref[0]
    o_ref[...] = jnp.where(bits >= lo, iw + jnp.float32(1.0), iw)


def kernel(difference, epoch, iteration, weight, initial_weight):
    del epoch, iteration, weight
    iw = jnp.asarray(initial_weight, jnp.float32).reshape((1,))
    return pl.pallas_call(
        _topk_mask_kernel,
        out_shape=jax.ShapeDtypeStruct((_ROWS, _COLS), jnp.float32),
        in_specs=[
            pl.BlockSpec(memory_space=pltpu.SMEM),
            pl.BlockSpec(memory_space=pltpu.VMEM),
        ],
        out_specs=pl.BlockSpec(memory_space=pltpu.VMEM),
    )(iw, difference)
